# baseline (speedup 1.0000x reference)
import functools

import jax
import jax.numpy as jnp
import numpy as np
from jax import lax
from jax.experimental import pallas as pl
from jax.experimental.pallas import tpu as pltpu
from jax.experimental.pallas import tpu_sc as plsc

PH = PW = 7
G = 2
NB = PH * PW
K = G * G * 4
J = NB * K
JPAD = 896
NW = 32
LANES = 16
BLK_R = 128


def _coord_kernel(hw_ref, prep_ref, idx_ref, wgt_ref):
    h = hw_ref[0]
    w = hw_ref[1]
    hf = h.astype(jnp.float32)
    wf = w.astype(jnp.float32)
    j = lax.broadcasted_iota(jnp.int32, (BLK_R, JPAD), 1)
    ph = (j // (PW * 16)).astype(jnp.float32)
    pw = ((j // 16) % PW).astype(jnp.float32)
    iy = ((j // 8) % 2).astype(jnp.float32)
    ix = ((j // 4) % 2).astype(jnp.float32)
    ny = (j // 2) % 2
    nx = j % 2

    cw = prep_ref[:, 0:1]
    ch = prep_ref[:, 1:2]
    rw = prep_ref[:, 2:3]
    rh = prep_ref[:, 3:4]
    cs = prep_ref[:, 4:5]
    sn = prep_ref[:, 5:6]
    base = prep_ref[:, 6:7]

    bin_h = rh / PH
    bin_w = rw / PW
    yy = -rh / 2.0 + (ph + (iy + 0.5) / G) * bin_h
    xx = -rw / 2.0 + (pw + (ix + 0.5) / G) * bin_w
    y = yy * cs - xx * sn + ch
    x = yy * sn + xx * cs + cw
    valid = (y >= -1.0) & (y <= hf) & (x >= -1.0) & (x <= wf)
    y = jnp.maximum(y, 0.0)
    x = jnp.maximum(x, 0.0)
    y_low = jnp.floor(y)
    x_low = jnp.floor(x)
    y_cond = y_low >= hf - 1.0
    x_cond = x_low >= wf - 1.0
    y_low = jnp.where(y_cond, hf - 1.0, y_low)
    x_low = jnp.where(x_cond, wf - 1.0, x_low)
    y_high = jnp.where(y_cond, hf - 1.0, y_low + 1.0)
    x_high = jnp.where(x_cond, wf - 1.0, x_low + 1.0)
    ly = jnp.where(y_cond, 0.0, y - y_low)
    lx = jnp.where(x_cond, 0.0, x - x_low)
    wy = jnp.where(ny == 1, ly, 1.0 - ly)
    wx = jnp.where(nx == 1, lx, 1.0 - lx)
    wgt_ref[...] = wy * wx * valid.astype(jnp.float32) * (1.0 / (G * G))
    ysel = jnp.where(ny == 1, y_high, y_low)
    xsel = jnp.where(nx == 1, x_high, x_low)
    idx_ref[...] = (base + ysel * wf + xsel).astype(jnp.int32)


BLK_H = 8


def _tr_kernel(in_ref, out_ref):
    out_ref[0] = jnp.transpose(in_ref[0], (1, 2, 0))


def _to_nhwc(x):
    n, c, h, w = x.shape
    nh = h // BLK_H
    out = pl.pallas_call(
        _tr_kernel,
        grid=(n, nh),
        in_specs=[pl.BlockSpec((1, c, BLK_H, w), lambda i, j: (i, 0, j, 0))],
        out_specs=pl.BlockSpec(
            (1, BLK_H, w, c), lambda i, j, _nh=nh: (i * _nh + j, 0, 0, 0)
        ),
        out_shape=jax.ShapeDtypeStruct((n * nh, BLK_H, w, c), jnp.float32),
    )(x)
    return out.reshape(n * h * w, c)


def _expand_rois(hw, prep, rpad):
    return pl.pallas_call(
        _coord_kernel,
        grid=(rpad // BLK_R,),
        in_specs=[
            pl.BlockSpec(memory_space=pltpu.SMEM),
            pl.BlockSpec((BLK_R, 8), lambda i: (i, 0)),
        ],
        out_specs=[
            pl.BlockSpec((BLK_R, JPAD), lambda i: (i, 0)),
            pl.BlockSpec((BLK_R, JPAD), lambda i: (i, 0)),
        ],
        out_shape=[
            jax.ShapeDtypeStruct((rpad, JPAD), jnp.int32),
            jax.ShapeDtypeStruct((rpad, JPAD), jnp.float32),
        ],
    )(hw, prep)


def _sc_align(table, idxs, wgts, c, rpad):
    rpw = rpad // NW
    cb_n = c // LANES

    mesh = plsc.VectorSubcoreMesh(core_axis_name="c", subcore_axis_name="s")
    chunk = 7 * K
    nchunk = NB // 7

    @functools.partial(
        pl.kernel,
        mesh=mesh,
        out_type=jax.ShapeDtypeStruct((rpad, NB, c), jnp.float32),
        scratch_types=[
            pltpu.VMEM((2 * JPAD,), jnp.int32),
            pltpu.VMEM((2 * JPAD,), jnp.float32),
            pltpu.VMEM((2, chunk, c), jnp.float32),
            pltpu.VMEM((2, NB, c), jnp.float32),
            pltpu.SemaphoreType.DMA,
            pltpu.SemaphoreType.DMA,
            pltpu.SemaphoreType.DMA,
        ],
    )
    def body(table_hbm, idx_hbm, wgt_hbm, out_hbm, idx_v, w_v, rows_v, stage_v, sem, osem, isem):
        cid = lax.axis_index("c")
        sid = lax.axis_index("s")
        wid = sid * 2 + cid
        r0 = wid * rpw

        pltpu.sync_copy(idx_hbm.at[r0], idx_v.at[pl.ds(0, JPAD)])
        pltpu.sync_copy(wgt_hbm.at[r0], w_v.at[pl.ds(0, JPAD)])
        pltpu.async_copy(
            table_hbm.at[idx_v.at[pl.ds(0, chunk)]], rows_v.at[0], sem
        )

        def roi_body(i, carry):
            r = r0 + i
            islot = lax.rem(i, 2)
            sslot = islot
            ibase = islot * JPAD
            nbase = (1 - islot) * JPAD

            @pl.when(i + 1 < rpw)
            def _prefetch_idx():
                pltpu.async_copy(idx_hbm.at[r + 1], idx_v.at[pl.ds(nbase, JPAD)], isem)
                pltpu.async_copy(wgt_hbm.at[r + 1], w_v.at[pl.ds(nbase, JPAD)], isem)

            @pl.when(i >= 2)
            def _drain_out():
                pltpu.make_async_copy(stage_v.at[sslot], out_hbm.at[r], osem).wait()

            def chunk_body(p, carry2):
                gslot = lax.rem(i * nchunk + p, 2)
                pltpu.make_async_copy(
                    table_hbm.at[idx_v.at[pl.ds(0, chunk)]], rows_v.at[gslot], sem
                ).wait()

                @pl.when(p + 1 < nchunk)
                def _prefetch():
                    pltpu.async_copy(
                        table_hbm.at[idx_v.at[pl.ds(ibase + (p + 1) * chunk, chunk)]],
                        rows_v.at[1 - gslot],
                        sem,
                    )

                @pl.when((p + 1 == nchunk) & (i + 1 < rpw))
                def _prefetch_next_roi():
                    pltpu.make_async_copy(
                        idx_hbm.at[r + 1], idx_v.at[pl.ds(nbase, JPAD)], isem
                    ).wait()
                    pltpu.make_async_copy(
                        wgt_hbm.at[r + 1], w_v.at[pl.ds(nbase, JPAD)], isem
                    ).wait()
                    pltpu.async_copy(
                        table_hbm.at[idx_v.at[pl.ds(nbase, chunk)]],
                        rows_v.at[1 - gslot],
                        sem,
                    )

                def bin_body(bb, carry3):
                    b = p * 7 + bb
                    w16 = w_v[pl.ds(ibase + b * K, LANES)]
                    wsp = [jnp.full((LANES,), w16[j], jnp.float32) for j in range(K)]

                    def cb_step(cb, carry4):
                        ch0 = cb * LANES
                        acc = wsp[0] * rows_v[gslot, bb * K, pl.ds(ch0, LANES)]
                        for j2 in range(1, K):
                            acc = acc + wsp[j2] * rows_v[gslot, bb * K + j2, pl.ds(ch0, LANES)]
                        stage_v[sslot, b, pl.ds(ch0, LANES)] = acc
                        return carry4

                    lax.fori_loop(0, cb_n, cb_step, 0)
                    return carry3

                lax.fori_loop(0, 7, bin_body, 0)
                return carry2

            lax.fori_loop(0, nchunk, chunk_body, 0)
            pltpu.async_copy(stage_v.at[sslot], out_hbm.at[r], osem)
            return carry

        lax.fori_loop(0, rpw, roi_body, 0)
        pltpu.make_async_copy(stage_v.at[0], out_hbm.at[r0], osem).wait()
        pltpu.make_async_copy(stage_v.at[1], out_hbm.at[r0], osem).wait()

    return body(table, idxs, wgts)


def kernel(input, rois):
    n, c, h, w = input.shape
    r = rois.shape[0]
    rpad = ((r + NW - 1) // NW) * NW
    rpad = max(((rpad + BLK_R - 1) // BLK_R) * BLK_R, BLK_R)

    table = _to_nhwc(input)

    scale = 0.25
    rp = jnp.concatenate([rois, jnp.zeros((rpad - r, 6), rois.dtype)], axis=0)
    theta = rp[:, 5] * (np.pi / 180.0)
    prep = jnp.stack(
        [
            rp[:, 1] * scale - 0.5,
            rp[:, 2] * scale - 0.5,
            rp[:, 3] * scale,
            rp[:, 4] * scale,
            jnp.cos(theta),
            jnp.sin(theta),
            rp[:, 0] * float(h * w),
            jnp.zeros_like(rp[:, 0]),
        ],
        axis=1,
    )
    hw = jnp.array([h, w], dtype=jnp.int32)

    idxs, wgts = _expand_rois(hw, prep, rpad)
    out = _sc_align(table, idxs, wgts, c, rpad)
    return jnp.transpose(out[:r], (0, 2, 1)).reshape(r, c, PH, PW)


# speedup vs baseline: 2.0797x; 2.0797x over previous
import functools

import jax
import jax.numpy as jnp
import numpy as np
from jax import lax
from jax.experimental import pallas as pl
from jax.experimental.pallas import tpu as pltpu
from jax.experimental.pallas import tpu_sc as plsc

PH = PW = 7
G = 2
NB = PH * PW
K = G * G * 4
J = NB * K
JPAD = 896
NW = 32
LANES = 16
BLK_R = 128


def _coord_kernel(hw_ref, prep_ref, idx_ref, wgt_ref):
    h = hw_ref[0]
    w = hw_ref[1]
    hf = h.astype(jnp.float32)
    wf = w.astype(jnp.float32)
    j = lax.broadcasted_iota(jnp.int32, (BLK_R, JPAD), 1)
    ph = (j // (PW * 16)).astype(jnp.float32)
    pw = ((j // 16) % PW).astype(jnp.float32)
    iy = ((j // 8) % 2).astype(jnp.float32)
    ix = ((j // 4) % 2).astype(jnp.float32)
    ny = (j // 2) % 2
    nx = j % 2

    cw = prep_ref[:, 0:1]
    ch = prep_ref[:, 1:2]
    rw = prep_ref[:, 2:3]
    rh = prep_ref[:, 3:4]
    cs = prep_ref[:, 4:5]
    sn = prep_ref[:, 5:6]
    base = prep_ref[:, 6:7]

    bin_h = rh / PH
    bin_w = rw / PW
    yy = -rh / 2.0 + (ph + (iy + 0.5) / G) * bin_h
    xx = -rw / 2.0 + (pw + (ix + 0.5) / G) * bin_w
    y = yy * cs - xx * sn + ch
    x = yy * sn + xx * cs + cw
    valid = (y >= -1.0) & (y <= hf) & (x >= -1.0) & (x <= wf)
    y = jnp.maximum(y, 0.0)
    x = jnp.maximum(x, 0.0)
    y_low = jnp.floor(y)
    x_low = jnp.floor(x)
    y_cond = y_low >= hf - 1.0
    x_cond = x_low >= wf - 1.0
    y_low = jnp.where(y_cond, hf - 1.0, y_low)
    x_low = jnp.where(x_cond, wf - 1.0, x_low)
    y_high = jnp.where(y_cond, hf - 1.0, y_low + 1.0)
    x_high = jnp.where(x_cond, wf - 1.0, x_low + 1.0)
    ly = jnp.where(y_cond, 0.0, y - y_low)
    lx = jnp.where(x_cond, 0.0, x - x_low)
    wy = jnp.where(ny == 1, ly, 1.0 - ly)
    wx = jnp.where(nx == 1, lx, 1.0 - lx)
    wgt_ref[...] = wy * wx * valid.astype(jnp.float32) * (1.0 / (G * G))
    ysel = jnp.where(ny == 1, y_high, y_low)
    xsel = jnp.where(nx == 1, x_high, x_low)
    idx_ref[...] = (base + ysel * wf + xsel).astype(jnp.int32)


BLK_H = 8


def _tr_kernel(in_ref, out_ref):
    out_ref[0] = jnp.transpose(in_ref[0], (1, 2, 0))


def _to_nhwc(x):
    n, c, h, w = x.shape
    nh = h // BLK_H
    out = pl.pallas_call(
        _tr_kernel,
        grid=(n, nh),
        in_specs=[pl.BlockSpec((1, c, BLK_H, w), lambda i, j: (i, 0, j, 0))],
        out_specs=pl.BlockSpec(
            (1, BLK_H, w, c), lambda i, j, _nh=nh: (i * _nh + j, 0, 0, 0)
        ),
        out_shape=jax.ShapeDtypeStruct((n * nh, BLK_H, w, c), jnp.float32),
    )(x)
    return out.reshape(n * h * w, c)


def _expand_rois(hw, prep, rpad):
    return pl.pallas_call(
        _coord_kernel,
        grid=(rpad // BLK_R,),
        in_specs=[
            pl.BlockSpec(memory_space=pltpu.SMEM),
            pl.BlockSpec((BLK_R, 8), lambda i: (i, 0)),
        ],
        out_specs=[
            pl.BlockSpec((BLK_R, JPAD), lambda i: (i, 0)),
            pl.BlockSpec((BLK_R, JPAD), lambda i: (i, 0)),
        ],
        out_shape=[
            jax.ShapeDtypeStruct((rpad, JPAD), jnp.int32),
            jax.ShapeDtypeStruct((rpad, JPAD), jnp.float32),
        ],
    )(hw, prep)


def _sc_align(table, idxs, wgts, c, rpad):
    rpw = rpad // NW
    cb_n = c // LANES

    mesh = plsc.VectorSubcoreMesh(core_axis_name="c", subcore_axis_name="s")
    chunk = 7 * K
    nchunk = NB // 7

    @functools.partial(
        pl.kernel,
        mesh=mesh,
        out_type=jax.ShapeDtypeStruct((rpad, NB, c), jnp.float32),
        scratch_types=[
            pltpu.VMEM((2 * JPAD,), jnp.int32),
            pltpu.VMEM((2 * JPAD,), jnp.float32),
            pltpu.VMEM((3, chunk, c), jnp.float32),
            pltpu.VMEM((2, NB, c), jnp.float32),
            pltpu.SemaphoreType.DMA,
            pltpu.SemaphoreType.DMA,
            pltpu.SemaphoreType.DMA,
            pltpu.SemaphoreType.DMA,
        ],
    )
    def body(table_hbm, idx_hbm, wgt_hbm, out_hbm, idx_v, w_v, rows_v, stage_v, sem, semb, osem, isem):
        cid = lax.axis_index("c")
        sid = lax.axis_index("s")
        wid = sid * 2 + cid
        r0 = wid * rpw

        pltpu.sync_copy(idx_hbm.at[r0], idx_v.at[pl.ds(0, JPAD)])
        pltpu.sync_copy(wgt_hbm.at[r0], w_v.at[pl.ds(0, JPAD)])
        pltpu.async_copy(
            table_hbm.at[idx_v.at[pl.ds(0, chunk)]], rows_v.at[0], sem
        )
        pltpu.async_copy(
            table_hbm.at[idx_v.at[pl.ds(chunk, chunk)]], rows_v.at[1], semb
        )

        def roi_body(i, carry):
            r = r0 + i
            islot = lax.rem(i, 2)
            sslot = islot
            ibase = islot * JPAD
            nbase = (1 - islot) * JPAD

            @pl.when(i + 1 < rpw)
            def _prefetch_idx():
                pltpu.async_copy(idx_hbm.at[r + 1], idx_v.at[pl.ds(nbase, JPAD)], isem)
                pltpu.async_copy(wgt_hbm.at[r + 1], w_v.at[pl.ds(nbase, JPAD)], isem)

            @pl.when(i >= 2)
            def _drain_out():
                pltpu.make_async_copy(stage_v.at[sslot], out_hbm.at[r], osem).wait()

            def chunk_body(p, carry2):
                t = i * nchunk + p
                gslot = lax.rem(t, 3)
                nslot = lax.rem(t + 2, 3)
                even = lax.rem(t, 2) == 0

                @pl.when(even)
                def _wait_e():
                    pltpu.make_async_copy(
                        table_hbm.at[idx_v.at[pl.ds(0, chunk)]], rows_v.at[gslot], sem
                    ).wait()

                @pl.when(jnp.logical_not(even))
                def _wait_o():
                    pltpu.make_async_copy(
                        table_hbm.at[idx_v.at[pl.ds(0, chunk)]], rows_v.at[gslot], semb
                    ).wait()

                def _issue(off, s):
                    pltpu.async_copy(
                        table_hbm.at[idx_v.at[pl.ds(off, chunk)]],
                        rows_v.at[nslot],
                        s,
                    )

                @pl.when(p < nchunk - 2)
                def _prefetch_same_roi():
                    off = ibase + (p + 2) * chunk

                    @pl.when(even)
                    def _ie():
                        _issue(off, sem)

                    @pl.when(jnp.logical_not(even))
                    def _io():
                        _issue(off, semb)

                @pl.when((p == nchunk - 2) & (i + 1 < rpw))
                def _prefetch_roi_c0():
                    pltpu.make_async_copy(
                        idx_hbm.at[r + 1], idx_v.at[pl.ds(nbase, JPAD)], isem
                    ).wait()
                    pltpu.make_async_copy(
                        wgt_hbm.at[r + 1], w_v.at[pl.ds(nbase, JPAD)], isem
                    ).wait()

                    @pl.when(even)
                    def _ie():
                        _issue(nbase, sem)

                    @pl.when(jnp.logical_not(even))
                    def _io():
                        _issue(nbase, semb)

                @pl.when((p == nchunk - 1) & (i + 1 < rpw))
                def _prefetch_roi_c1():
                    @pl.when(even)
                    def _ie():
                        _issue(nbase + chunk, sem)

                    @pl.when(jnp.logical_not(even))
                    def _io():
                        _issue(nbase + chunk, semb)

                def bin_body(bb, carry3):
                    b = p * 7 + bb
                    w16 = w_v[pl.ds(ibase + b * K, LANES)]
                    wsp = [jnp.full((LANES,), w16[j], jnp.float32) for j in range(K)]

                    def cb_step(cb, carry4):
                        ch0 = cb * LANES
                        acc = wsp[0] * rows_v[gslot, bb * K, pl.ds(ch0, LANES)]
                        for j2 in range(1, K):
                            acc = acc + wsp[j2] * rows_v[gslot, bb * K + j2, pl.ds(ch0, LANES)]
                        stage_v[sslot, b, pl.ds(ch0, LANES)] = acc
                        return carry4

                    lax.fori_loop(0, cb_n, cb_step, 0)
                    return carry3

                lax.fori_loop(0, 7, bin_body, 0)
                return carry2

            lax.fori_loop(0, nchunk, chunk_body, 0)
            pltpu.async_copy(stage_v.at[sslot], out_hbm.at[r], osem)
            return carry

        lax.fori_loop(0, rpw, roi_body, 0)
        pltpu.make_async_copy(stage_v.at[0], out_hbm.at[r0], osem).wait()
        pltpu.make_async_copy(stage_v.at[1], out_hbm.at[r0], osem).wait()

    return body(table, idxs, wgts)


def kernel(input, rois):
    n, c, h, w = input.shape
    r = rois.shape[0]
    rpad = ((r + NW - 1) // NW) * NW
    rpad = max(((rpad + BLK_R - 1) // BLK_R) * BLK_R, BLK_R)

    table = jnp.transpose(input, (0, 2, 3, 1)).reshape(n * h * w, c)

    scale = 0.25
    rp = jnp.concatenate([rois, jnp.zeros((rpad - r, 6), rois.dtype)], axis=0)
    theta = rp[:, 5] * (np.pi / 180.0)
    prep = jnp.stack(
        [
            rp[:, 1] * scale - 0.5,
            rp[:, 2] * scale - 0.5,
            rp[:, 3] * scale,
            rp[:, 4] * scale,
            jnp.cos(theta),
            jnp.sin(theta),
            rp[:, 0] * float(h * w),
            jnp.zeros_like(rp[:, 0]),
        ],
        axis=1,
    )
    hw = jnp.array([h, w], dtype=jnp.int32)

    idxs, wgts = _expand_rois(hw, prep, rpad)
    out = _sc_align(table, idxs, wgts, c, rpad)
    return jnp.transpose(out[:r], (0, 2, 1)).reshape(r, c, PH, PW)
